# baseline (speedup 1.0000x reference)
import jax
import jax.numpy as jnp
from jax.experimental import pallas as pl
from jax.experimental.pallas import tpu as pltpu


def _fused_head_kernel(x_hbm, w1_ref, b1_ref, w2_ref, b2_ref, w3_ref, b3_ref,
                       o_ref, acc_ref, xpad_ref, pad2_ref, sem):
    b = pl.program_id(0)
    t = pl.program_id(1)
    kd = t // 3
    kh = t % 3

    @pl.when(t == 0)
    def _():
        acc_ref[...] = jnp.zeros_like(acc_ref)
        for bb in range(8):
            for jd in range(2):
                for jh in range(2):
                    for jw in range(2):
                        pltpu.make_async_copy(
                            x_hbm.at[b * 8 + bb, :, jd, :, jh, :, jw, :],
                            xpad_ref.at[1 - jd, 1 - jh, 1 - jw,
                                        pl.ds(jd, 4), pl.ds(jh, 4),
                                        pl.ds(jw, 4), bb, :],
                            sem).start()
        xpad_ref[0, :, :, 0, :, :, :, :] = jnp.zeros_like(
            xpad_ref[0, :, :, 0, :, :, :, :])
        xpad_ref[1, :, :, 4, :, :, :, :] = jnp.zeros_like(
            xpad_ref[1, :, :, 4, :, :, :, :])
        xpad_ref[:, 0, :, :, 0, :, :, :] = jnp.zeros_like(
            xpad_ref[:, 0, :, :, 0, :, :, :])
        xpad_ref[:, 1, :, :, 4, :, :, :] = jnp.zeros_like(
            xpad_ref[:, 1, :, :, 4, :, :, :])
        xpad_ref[:, :, 0, :, :, 0, :, :] = jnp.zeros_like(
            xpad_ref[:, :, 0, :, :, 0, :, :])
        xpad_ref[:, :, 1, :, :, 4, :, :] = jnp.zeros_like(
            xpad_ref[:, :, 1, :, :, 4, :, :])
        for _ in range(64):
            pltpu.make_async_copy(
                x_hbm.at[0, :, 0, :, 0, :, 0, :],
                xpad_ref.at[0, 0, 0, pl.ds(0, 4), pl.ds(0, 4), pl.ds(0, 4),
                            0, :],
                sem).wait()

    for kw in range(3):
        a = xpad_ref[kd % 2, kh % 2, kw % 2,
                     pl.ds(kd // 2, 4), pl.ds(kh // 2, 4), pl.ds(kw // 2, 4),
                     :, :]
        a = a.reshape(512, 768).astype(jnp.bfloat16)
        acc_ref[...] += jnp.dot(a, w1_ref[768 * kw:768 * (kw + 1), :],
                                preferred_element_type=jnp.float32)

    @pl.when(t == 8)
    def _():
        h1 = jnp.maximum(acc_ref[...] + b1_ref[...], 0.0).astype(jnp.bfloat16)
        pad2_ref[...] = jnp.zeros_like(pad2_ref)
        pad2_ref[1:5, 1:5, 1:5, :, :] = h1.reshape(4, 4, 4, 8, 512)

        pv = pad2_ref[...].reshape(3, 2, 3, 2, 3, 2, 8, 512)
        acc2 = jnp.zeros((64, 256), jnp.float32)
        for dz in range(3):
            for dy in range(3):
                for dx in range(3):
                    aa = pv[dz // 2:dz // 2 + 2, dz % 2,
                            dy // 2:dy // 2 + 2, dy % 2,
                            dx // 2:dx // 2 + 2, dx % 2, :, :]
                    ti = dz * 9 + dy * 3 + dx
                    acc2 += jnp.dot(aa.reshape(64, 512),
                                    w2_ref[512 * ti:512 * (ti + 1), :],
                                    preferred_element_type=jnp.float32)
        h2 = jnp.maximum(acc2 + b2_ref[...], 0.0).astype(jnp.bfloat16)
        h2 = h2.reshape(2, 2, 2, 8, 256)

        acc3 = jnp.zeros((8, 128), jnp.float32)
        for dz in range(1, 3):
            for dy in range(1, 3):
                for dx in range(1, 3):
                    ti = dz * 9 + dy * 3 + dx
                    acc3 += jnp.dot(h2[dz - 1, dy - 1, dx - 1],
                                    w3_ref[256 * ti:256 * (ti + 1), :],
                                    preferred_element_type=jnp.float32)
        emb = acc3 + b3_ref[...]

        ss = jnp.sum(emb * emb, axis=1, keepdims=True)
        o_ref[...] = emb * jax.lax.rsqrt(jnp.maximum(ss, 1e-24))


def kernel(x_raw, embed_last, wmat0, bias0, wmat1, bias1, wmat2, bias2):
    del x_raw

    x = embed_last.transpose(0, 2, 3, 4, 1)
    x = x.reshape(16, 4, 2, 4, 2, 4, 2, 768)

    return pl.pallas_call(
        _fused_head_kernel,
        out_shape=jax.ShapeDtypeStruct((16, 128), jnp.float32),
        grid=(2, 9),
        in_specs=[
            pl.BlockSpec(memory_space=pl.ANY),
            pl.BlockSpec((2304, 512), lambda b, t: (t, 0)),
            pl.BlockSpec((1, 512), lambda b, t: (0, 0)),
            pl.BlockSpec((13824, 256), lambda b, t: (0, 0)),
            pl.BlockSpec((1, 256), lambda b, t: (0, 0)),
            pl.BlockSpec((6912, 128), lambda b, t: (0, 0)),
            pl.BlockSpec((1, 128), lambda b, t: (0, 0)),
        ],
        out_specs=pl.BlockSpec((8, 128), lambda b, t: (b, 0)),
        scratch_shapes=[
            pltpu.VMEM((512, 512), jnp.float32),
            pltpu.VMEM((2, 2, 2, 5, 5, 5, 8, 768), jnp.float32),
            pltpu.VMEM((6, 6, 6, 8, 512), jnp.bfloat16),
            pltpu.SemaphoreType.DMA,
        ],
        compiler_params=pltpu.CompilerParams(
            dimension_semantics=("parallel", "arbitrary"),
            vmem_limit_bytes=56 * 1024 * 1024),
        name="fused_get_embed_head",
    )(x, wmat0, bias0, wmat1, bias1, wmat2, bias2)


# speedup vs baseline: 2.4446x; 2.4446x over previous
import jax
import jax.numpy as jnp
from jax.experimental import pallas as pl
from jax.experimental.pallas import tpu as pltpu

_W_SRC = ((0, 4, 1), (3, 5, 0))


def _fused_head_kernel(x_hbm, w1_ref, b1_ref, w2_ref, b2_ref, w3_ref, b3_ref,
                       o_ref, acc_ref, xpad_ref, pad2_ref, sem):
    h = pl.program_id(0)
    t = pl.program_id(1)
    kd = t // 3
    kh = t % 3

    @pl.when(t == 0)
    def _():
        @pl.when(h == 0)
        def _():
            acc_ref[...] = jnp.zeros_like(acc_ref)
        for hh in range(2):
            src_w, w_len, dst_w = _W_SRC[hh]

            @pl.when(h == hh)
            def _():
                for bb in range(16):
                    for jd in range(2):
                        for jh in range(2):
                            pltpu.make_async_copy(
                                x_hbm.at[bb, :, jd, :, jh,
                                         pl.ds(src_w, w_len), :],
                                xpad_ref.at[1 - jd, 1 - jh,
                                            pl.ds(jd, 4), pl.ds(jh, 4),
                                            pl.ds(dst_w, w_len), bb, :],
                                sem).start()
        xpad_ref[0, :, 0, :, :, :, :] = jnp.zeros_like(
            xpad_ref[0, :, 0, :, :, :, :])
        xpad_ref[1, :, 4, :, :, :, :] = jnp.zeros_like(
            xpad_ref[1, :, 4, :, :, :, :])
        xpad_ref[:, 0, :, 0, :, :, :] = jnp.zeros_like(
            xpad_ref[:, 0, :, 0, :, :, :])
        xpad_ref[:, 1, :, 4, :, :, :] = jnp.zeros_like(
            xpad_ref[:, 1, :, 4, :, :, :])

        @pl.when(h == 0)
        def _():
            xpad_ref[:, :, :, :, 0, :, :] = jnp.zeros_like(
                xpad_ref[:, :, :, :, 0, :, :])
        for hh in range(2):
            src_w, w_len, dst_w = _W_SRC[hh]

            @pl.when(h == hh)
            def _():
                for _ in range(64):
                    pltpu.make_async_copy(
                        x_hbm.at[0, :, 0, :, 0, pl.ds(src_w, w_len), :],
                        xpad_ref.at[0, 0, pl.ds(0, 4), pl.ds(0, 4),
                                    pl.ds(dst_w, w_len), 0, :],
                        sem).wait()

    for kw in range(3):
        pieces = []
        for owl in range(2):
            p = xpad_ref[kd % 2, kh % 2,
                         pl.ds(kd // 2, 4), pl.ds(kh // 2, 4),
                         2 * owl + kw, :, :]
            pieces.append(p.reshape(256, 768))
        a = jnp.concatenate(pieces, axis=0).astype(jnp.bfloat16)
        acc_ref[pl.ds(512 * h, 512), :] += jnp.dot(
            a, w1_ref[768 * kw:768 * (kw + 1), :],
            preferred_element_type=jnp.float32)

    @pl.when((t == 8) & (h == 1))
    def _():
        h1 = jnp.maximum(acc_ref[...] + b1_ref[...], 0.0).astype(jnp.bfloat16)
        pad2_ref[...] = jnp.zeros_like(pad2_ref)
        pad2_ref[1:5, 1:5, 1:5, :, :] = h1.reshape(4, 4, 4, 16, 512)

        pv = pad2_ref[...].reshape(3, 2, 3, 2, 3, 2, 16, 512)
        acc2 = jnp.zeros((128, 256), jnp.float32)
        for dz in range(3):
            for dy in range(3):
                for dx in range(3):
                    aa = pv[dx // 2:dx // 2 + 2, dx % 2,
                            dz // 2:dz // 2 + 2, dz % 2,
                            dy // 2:dy // 2 + 2, dy % 2, :, :]
                    ti = dz * 9 + dy * 3 + dx
                    acc2 += jnp.dot(aa.reshape(128, 512),
                                    w2_ref[512 * ti:512 * (ti + 1), :],
                                    preferred_element_type=jnp.float32)
        h2 = jnp.maximum(acc2 + b2_ref[...], 0.0).astype(jnp.bfloat16)
        h2 = h2.reshape(2, 2, 2, 16, 256)

        acc3 = jnp.zeros((16, 128), jnp.float32)
        for dz in range(1, 3):
            for dy in range(1, 3):
                for dx in range(1, 3):
                    ti = dz * 9 + dy * 3 + dx
                    acc3 += jnp.dot(h2[dx - 1, dz - 1, dy - 1],
                                    w3_ref[256 * ti:256 * (ti + 1), :],
                                    preferred_element_type=jnp.float32)
        emb = acc3 + b3_ref[...]

        ss = jnp.sum(emb * emb, axis=1, keepdims=True)
        o_ref[...] = emb * jax.lax.rsqrt(jnp.maximum(ss, 1e-24))


def kernel(x_raw, embed_last, wmat0, bias0, wmat1, bias1, wmat2, bias2):
    del x_raw

    x = embed_last.transpose(0, 2, 3, 4, 1)
    x = x.reshape(16, 4, 2, 4, 2, 8, 768)

    return pl.pallas_call(
        _fused_head_kernel,
        out_shape=jax.ShapeDtypeStruct((16, 128), jnp.float32),
        grid=(2, 9),
        in_specs=[
            pl.BlockSpec(memory_space=pl.ANY),
            pl.BlockSpec((2304, 512), lambda h, t: (t, 0)),
            pl.BlockSpec((1, 512), lambda h, t: (0, 0)),
            pl.BlockSpec((13824, 256), lambda h, t: (0, 0)),
            pl.BlockSpec((1, 256), lambda h, t: (0, 0)),
            pl.BlockSpec((6912, 128), lambda h, t: (0, 0)),
            pl.BlockSpec((1, 128), lambda h, t: (0, 0)),
        ],
        out_specs=pl.BlockSpec((16, 128), lambda h, t: (0, 0)),
        scratch_shapes=[
            pltpu.VMEM((1024, 512), jnp.float32),
            pltpu.VMEM((2, 2, 5, 5, 5, 16, 768), jnp.float32),
            pltpu.VMEM((6, 6, 6, 16, 512), jnp.bfloat16),
            pltpu.SemaphoreType.DMA,
        ],
        compiler_params=pltpu.CompilerParams(
            dimension_semantics=("arbitrary", "arbitrary"),
            vmem_limit_bytes=56 * 1024 * 1024),
        name="fused_get_embed_head",
    )(x, wmat0, bias0, wmat1, bias1, wmat2, bias2)


# speedup vs baseline: 2.4464x; 1.0007x over previous
import jax
import jax.numpy as jnp
from jax.experimental import pallas as pl
from jax.experimental.pallas import tpu as pltpu

_W_SRC = ((0, 4, 1), (3, 5, 0))


def _fused_head_kernel(x_hbm, w1_ref, b1_ref, w2_ref, b2_ref, w3_ref, b3_ref,
                       o_ref, acc_ref, xpad_ref, pad2_ref, sem):
    h = pl.program_id(0)
    t = pl.program_id(1)
    kd = t // 3
    kh = t % 3

    @pl.when(t == 0)
    def _():
        @pl.when(h == 0)
        def _():
            acc_ref[...] = jnp.zeros_like(acc_ref)
        for hh in range(2):
            src_w, w_len, dst_w = _W_SRC[hh]

            @pl.when(h == hh)
            def _():
                for bb in range(16):
                    for jd in range(2):
                        for jh in range(2):
                            pltpu.make_async_copy(
                                x_hbm.at[bb, :, jd, :, jh,
                                         pl.ds(src_w, w_len), :],
                                xpad_ref.at[1 - jd, 1 - jh,
                                            pl.ds(jd, 4), pl.ds(jh, 4),
                                            pl.ds(dst_w, w_len), bb, :],
                                sem).start()
        xpad_ref[0, :, 0, :, :, :, :] = jnp.zeros_like(
            xpad_ref[0, :, 0, :, :, :, :])
        xpad_ref[1, :, 4, :, :, :, :] = jnp.zeros_like(
            xpad_ref[1, :, 4, :, :, :, :])
        xpad_ref[:, 0, :, 0, :, :, :] = jnp.zeros_like(
            xpad_ref[:, 0, :, 0, :, :, :])
        xpad_ref[:, 1, :, 4, :, :, :] = jnp.zeros_like(
            xpad_ref[:, 1, :, 4, :, :, :])

        @pl.when(h == 0)
        def _():
            xpad_ref[:, :, :, :, 0, :, :] = jnp.zeros_like(
                xpad_ref[:, :, :, :, 0, :, :])
        for hh in range(2):
            src_w, w_len, dst_w = _W_SRC[hh]

            @pl.when(h == hh)
            def _():
                for _ in range(64):
                    pltpu.make_async_copy(
                        x_hbm.at[0, :, 0, :, 0, pl.ds(src_w, w_len), :],
                        xpad_ref.at[0, 0, pl.ds(0, 4), pl.ds(0, 4),
                                    pl.ds(dst_w, w_len), 0, :],
                        sem).wait()

    cols = []
    for kw in range(3):
        pieces = []
        for owl in range(2):
            p = xpad_ref[kd % 2, kh % 2,
                         pl.ds(kd // 2, 4), pl.ds(kh // 2, 4),
                         2 * owl + kw, :, :]
            pieces.append(p.reshape(256, 768))
        cols.append(jnp.concatenate(pieces, axis=0).astype(jnp.bfloat16))
    a = jnp.concatenate(cols, axis=1)
    acc_ref[pl.ds(512 * h, 512), :] += jnp.dot(
        a, w1_ref[...], preferred_element_type=jnp.float32)

    @pl.when((t == 8) & (h == 1))
    def _():
        h1 = jnp.maximum(acc_ref[...] + b1_ref[...], 0.0).astype(jnp.bfloat16)
        pad2_ref[...] = jnp.zeros_like(pad2_ref)
        pad2_ref[1:5, 1:5, 1:5, :, :] = h1.reshape(4, 4, 4, 16, 512)

        pv = pad2_ref[...].reshape(3, 2, 3, 2, 3, 2, 16, 512)
        acc2 = jnp.zeros((128, 256), jnp.float32)
        for dz in range(3):
            for dy in range(3):
                for dx in range(3):
                    aa = pv[dx // 2:dx // 2 + 2, dx % 2,
                            dz // 2:dz // 2 + 2, dz % 2,
                            dy // 2:dy // 2 + 2, dy % 2, :, :]
                    ti = dz * 9 + dy * 3 + dx
                    acc2 += jnp.dot(aa.reshape(128, 512),
                                    w2_ref[512 * ti:512 * (ti + 1), :],
                                    preferred_element_type=jnp.float32)
        h2 = jnp.maximum(acc2 + b2_ref[...], 0.0).astype(jnp.bfloat16)
        h2 = h2.reshape(2, 2, 2, 16, 256)

        acc3 = jnp.zeros((16, 128), jnp.float32)
        for dz in range(1, 3):
            for dy in range(1, 3):
                for dx in range(1, 3):
                    ti = dz * 9 + dy * 3 + dx
                    acc3 += jnp.dot(h2[dx - 1, dz - 1, dy - 1],
                                    w3_ref[256 * ti:256 * (ti + 1), :],
                                    preferred_element_type=jnp.float32)
        emb = acc3 + b3_ref[...]

        ss = jnp.sum(emb * emb, axis=1, keepdims=True)
        o_ref[...] = emb * jax.lax.rsqrt(jnp.maximum(ss, 1e-24))


def kernel(x_raw, embed_last, wmat0, bias0, wmat1, bias1, wmat2, bias2):
    del x_raw

    x = embed_last.transpose(0, 2, 3, 4, 1)
    x = x.reshape(16, 4, 2, 4, 2, 8, 768)

    return pl.pallas_call(
        _fused_head_kernel,
        out_shape=jax.ShapeDtypeStruct((16, 128), jnp.float32),
        grid=(2, 9),
        in_specs=[
            pl.BlockSpec(memory_space=pl.ANY),
            pl.BlockSpec((2304, 512), lambda h, t: (t, 0)),
            pl.BlockSpec((1, 512), lambda h, t: (0, 0)),
            pl.BlockSpec((13824, 256), lambda h, t: (0, 0)),
            pl.BlockSpec((1, 256), lambda h, t: (0, 0)),
            pl.BlockSpec((6912, 128), lambda h, t: (0, 0)),
            pl.BlockSpec((1, 128), lambda h, t: (0, 0)),
        ],
        out_specs=pl.BlockSpec((16, 128), lambda h, t: (0, 0)),
        scratch_shapes=[
            pltpu.VMEM((1024, 512), jnp.float32),
            pltpu.VMEM((2, 2, 5, 5, 5, 16, 768), jnp.float32),
            pltpu.VMEM((6, 6, 6, 16, 512), jnp.bfloat16),
            pltpu.SemaphoreType.DMA,
        ],
        compiler_params=pltpu.CompilerParams(
            dimension_semantics=("arbitrary", "arbitrary"),
            vmem_limit_bytes=56 * 1024 * 1024),
        name="fused_get_embed_head",
    )(x, wmat0, bias0, wmat1, bias1, wmat2, bias2)
